# Initial kernel scaffold; baseline (speedup 1.0000x reference)
import jax
import jax.numpy as jnp
from jax.experimental import pallas as pl


def kernel(x, edge_index, batch, W1, b1, W2, b2):
    raise NotImplementedError("write your pallas kernel here")


# speedup vs baseline: 9.4250x; 9.4250x over previous
import functools

import jax
import jax.numpy as jnp
from jax import lax
from jax.experimental import pallas as pl
from jax.experimental.pallas import tpu as pltpu
from jax.experimental.pallas import tpu_sc as plsc

N = 10000
E = 320000
D = 128
B = 64

NC = 2
NS = 16
NW = NC * NS
CHUNK = 128
NCHUNK = 80
EPT = NCHUNK * CHUNK
E_PAD = NW * EPT
NPAD = 10112
RPT = NPAD // NS

_mesh = plsc.VectorSubcoreMesh(
    core_axis_name="c", subcore_axis_name="s", num_cores=NC, num_subcores=NS)

_sc_params = pltpu.CompilerParams(use_tc_tiling_on_sc=False)

_f32 = jnp.float32


@functools.partial(
    pl.kernel,
    out_type=[jax.ShapeDtypeStruct((NPAD, 16), _f32),
              jax.ShapeDtypeStruct((NPAD, 16), _f32)],
    mesh=_mesh,
    scratch_types=[
        pltpu.VMEM((NCHUNK, CHUNK), jnp.int32),
        pltpu.VMEM((CHUNK, 16), _f32),
        pltpu.VMEM((RPT, 16), _f32),
        pltpu.VMEM_SHARED((NPAD, 16), _f32),
    ],
    compiler_params=_sc_params,
)
def _sc_degree(dst_hbm, out0, out1, dst_v, ones_v, zb_v, acc):
    c = lax.axis_index("c")
    s = lax.axis_index("s")
    wid = s * NC + c
    ones16 = jnp.ones((16,), _f32)
    zeros16 = jnp.zeros((16,), _f32)

    def fill_ones(i, _):
        ones_v[i] = ones16
        return 0
    lax.fori_loop(0, CHUNK, fill_ones, 0)

    def fill_zeros(i, _):
        zb_v[i] = zeros16
        return 0
    lax.fori_loop(0, RPT, fill_zeros, 0)

    pltpu.sync_copy(zb_v, acc.at[pl.ds(s * RPT, RPT)])
    pltpu.sync_copy(dst_hbm.at[wid], dst_v)
    plsc.subcore_barrier()

    def body(j, _):
        pltpu.sync_copy(ones_v, acc.at[dst_v.at[j]], add=True)
        return 0
    lax.fori_loop(0, NCHUNK, body, 0)

    plsc.subcore_barrier()

    @pl.when(c == 0)
    def _():
        pltpu.sync_copy(acc.at[pl.ds(s * RPT, RPT)], out0.at[pl.ds(s * RPT, RPT)])

    @pl.when(c == 1)
    def _():
        pltpu.sync_copy(acc.at[pl.ds(s * RPT, RPT)], out1.at[pl.ds(s * RPT, RPT)])


@functools.partial(
    pl.kernel,
    out_type=[jax.ShapeDtypeStruct((NPAD, D), _f32),
              jax.ShapeDtypeStruct((NPAD, D), _f32)],
    mesh=_mesh,
    scratch_types=[
        pltpu.VMEM((NCHUNK, CHUNK), jnp.int32),
        pltpu.VMEM((NCHUNK, CHUNK), jnp.int32),
        pltpu.VMEM((CHUNK, D), _f32),
        pltpu.VMEM_SHARED((NPAD, D), _f32),
        pltpu.SemaphoreType.DMA,
    ],
    compiler_params=_sc_params,
)
def _sc_scatter(g_hbm, src_hbm, dst_hbm, out0, out1,
                src_v, dst_v, rows_v, acc, gsem):
    c = lax.axis_index("c")
    s = lax.axis_index("s")
    wid = s * NC + c
    zeros16 = jnp.zeros((16,), _f32)

    def zrow(r, _):
        def zcol(k, _2):
            rows_v[r, pl.ds(k * 16, 16)] = zeros16
            return 0
        lax.fori_loop(0, D // 16, zcol, 0)
        return 0
    lax.fori_loop(0, CHUNK, zrow, 0)

    for k in range(RPT // CHUNK):
        pltpu.sync_copy(rows_v, acc.at[pl.ds(s * RPT + k * CHUNK, CHUNK)])
    rem = RPT % CHUNK
    pltpu.sync_copy(rows_v.at[pl.ds(0, rem)],
                    acc.at[pl.ds(s * RPT + (RPT // CHUNK) * CHUNK, rem)])

    pltpu.sync_copy(src_hbm.at[wid], src_v)
    pltpu.sync_copy(dst_hbm.at[wid], dst_v)
    plsc.subcore_barrier()

    def body(j, _):
        pltpu.async_copy(g_hbm.at[src_v.at[j]], rows_v, gsem).wait()
        pltpu.sync_copy(rows_v, acc.at[dst_v.at[j]], add=True)
        return 0
    lax.fori_loop(0, NCHUNK, body, 0)

    plsc.subcore_barrier()

    @pl.when(c == 0)
    def _():
        pltpu.sync_copy(acc.at[pl.ds(s * RPT, RPT)], out0.at[pl.ds(s * RPT, RPT)])

    @pl.when(c == 1)
    def _():
        pltpu.sync_copy(acc.at[pl.ds(s * RPT, RPT)], out1.at[pl.ds(s * RPT, RPT)])


RB = 1000
_HI = jax.lax.Precision.HIGHEST


def _tc1_body(d0, d1, xr, w1, g1o, dvo):
    deg = 1.0 + d0[:, :1] + d1[:, :1]
    dv = jax.lax.rsqrt(deg)
    h = jnp.dot(xr[...], w1[...], preferred_element_type=_f32, precision=_HI)
    g1o[...] = h * dv
    dvo[...] = dv


def _tc2_body(a0, a1, g1r, dv, b1r, w2, g2o):
    z = (a0[...] + a1[...] + g1r[...]) * dv[...] + b1r[...]
    z = jnp.maximum(z, 0.0)
    g2o[...] = jnp.dot(z, w2[...], preferred_element_type=_f32,
                       precision=_HI) * dv[...]


def _tc3_body(a0, a1, g2r, dv, b2r, batch_r, out_ref, acc, cnt):
    i = pl.program_id(0)

    @pl.when(i == 0)
    def _():
        acc[...] = jnp.zeros_like(acc)
        cnt[...] = jnp.zeros_like(cnt)

    h = (a0[...] + a1[...] + g2r[...]) * dv[...] + b2r[...]
    bvec = batch_r[0]
    onehot = (lax.broadcasted_iota(jnp.int32, (B, RB), 0) == bvec).astype(_f32)
    acc[...] += jnp.dot(onehot, h, preferred_element_type=_f32, precision=_HI)
    cnt[...] += jnp.sum(onehot, axis=1, keepdims=True)

    @pl.when(i == pl.num_programs(0) - 1)
    def _():
        out_ref[...] = acc[...] / jnp.maximum(cnt[...], 1.0)


def _row_spec(w):
    return pl.BlockSpec((RB, w), lambda i: (i, 0))


def _const_spec(shape):
    return pl.BlockSpec(shape, lambda i: (0, 0))


def kernel(x, edge_index, batch, W1, b1, W2, b2):
    src = edge_index[0]
    dst = edge_index[1]
    pad = E_PAD - E
    src_p = jnp.concatenate([src, jnp.zeros((pad,), jnp.int32)])
    dst_p = jnp.concatenate([dst, jnp.full((pad,), N, jnp.int32)])
    src_c = src_p.reshape(NW, NCHUNK, CHUNK)
    dst_c = dst_p.reshape(NW, NCHUNK, CHUNK)
    b1r = b1.reshape(1, D)
    b2r = b2.reshape(1, D)
    batch_r = batch.reshape(N // RB, 1, RB)

    degp0, degp1 = _sc_degree(dst_c)

    g1, dinv = pl.pallas_call(
        _tc1_body,
        grid=(N // RB,),
        in_specs=[_row_spec(16), _row_spec(16), _row_spec(D), _const_spec((D, D))],
        out_specs=[_row_spec(D), _row_spec(1)],
        out_shape=[jax.ShapeDtypeStruct((N, D), _f32),
                   jax.ShapeDtypeStruct((N, 1), _f32)],
    )(degp0, degp1, x, W1)

    a0, a1 = _sc_scatter(g1, src_c, dst_c)

    g2 = pl.pallas_call(
        _tc2_body,
        grid=(N // RB,),
        in_specs=[_row_spec(D), _row_spec(D), _row_spec(D), _row_spec(1),
                  _const_spec((1, D)), _const_spec((D, D))],
        out_specs=_row_spec(D),
        out_shape=jax.ShapeDtypeStruct((N, D), _f32),
    )(a0, a1, g1, dinv, b1r, W2)

    c0, c1 = _sc_scatter(g2, src_c, dst_c)

    out = pl.pallas_call(
        _tc3_body,
        grid=(N // RB,),
        in_specs=[_row_spec(D), _row_spec(D), _row_spec(D), _row_spec(1),
                  _const_spec((1, D)),
                  pl.BlockSpec((1, 1, RB), lambda i: (i, 0, 0))],
        out_specs=_const_spec((B, D)),
        out_shape=jax.ShapeDtypeStruct((B, D), _f32),
        scratch_shapes=[pltpu.VMEM((B, D), _f32), pltpu.VMEM((B, D), _f32)],
    )(c0, c1, g2, dinv, b2r, batch_r)

    return out
